# baseline (speedup 1.0000x reference)
import functools
import numpy as np
import jax
import jax.numpy as jnp
from jax import lax
from jax.experimental import pallas as pl
from jax.experimental.pallas import tpu as pltpu
from jax.experimental.pallas import tpu_sc as plsc

D = 64
C = 2
H = 4
DH = 16
NCLS = 8
N = 10000
E = 160000
VOCAB = 1000

NWORK = 32
NPAD = 10240
ECH = 32
CPB = 8
NCHUNK = 160
NCH = 64
EPAD = NWORK * NCHUNK * ECH
ROW = 144


def _pos_encoding():
    pos = np.arange(C)[:, None].astype(np.float64)
    i = np.arange(D)[None, :].astype(np.float64)
    angle = pos / np.power(10000.0, (2.0 * (i // 2)) / D)
    pe = np.where(i % 2 == 0, np.sin(angle), np.cos(angle))
    return jnp.asarray(pe, dtype=jnp.float32)


def _tables_body(eu, ei, wq, wk, wv, pe, qt, kt, vt):
    for c in range(C):
        xu = eu[c] + pe[c][None, :]
        xi = ei[c] + pe[c][None, :]
        qt[c * VOCAB:(c + 1) * VOCAB, :] = jnp.dot(
            xu, wq[...], preferred_element_type=jnp.float32)
        kt[c * VOCAB:(c + 1) * VOCAB, :] = jnp.dot(
            xi, wk[...], preferred_element_type=jnp.float32)
        vt[c * VOCAB:(c + 1) * VOCAB, :] = jnp.dot(
            xi, wv[...], preferred_element_type=jnp.float32)


def _make_tables(emb_users, emb_items, Wq, Wk, Wv, pe):
    out = [jax.ShapeDtypeStruct((C * VOCAB, D), jnp.float32)] * 3
    return pl.pallas_call(_tables_body, out_shape=out)(
        emb_users, emb_items, Wq, Wk, Wv, pe)


def _stage1_body(qtab, ktab, vtab, iu0, iu1, ii0, ii1, tq_out, ts_out,
                 vi0, vi1, vi2, vi3, ga, gb, gc, gd, ge, gf,
                 tq_pack, ts_pack, sem):
    cid = lax.axis_index("c")
    sid = lax.axis_index("s")
    wid = sid * 2 + cid
    base = wid * (NPAD // NWORK)

    def chunk(j, _):
        b = base + j * NCH
        pltpu.sync_copy(iu0.at[pl.ds(b, NCH)], vi0)
        pltpu.sync_copy(iu1.at[pl.ds(b, NCH)], vi1)
        pltpu.sync_copy(ii0.at[pl.ds(b, NCH)], vi2)
        pltpu.sync_copy(ii1.at[pl.ds(b, NCH)], vi3)
        pltpu.async_copy(qtab.at[vi0], ga, sem).wait()
        pltpu.async_copy(qtab.at[vi1], gb, sem).wait()
        pltpu.async_copy(ktab.at[vi2], gc, sem).wait()
        pltpu.async_copy(ktab.at[vi3], gd, sem).wait()
        pltpu.async_copy(vtab.at[vi2], ge, sem).wait()
        pltpu.async_copy(vtab.at[vi3], gf, sem).wait()

        def row(r, _):
            for k in range(4):
                sl = pl.ds(k * DH, DH)
                tq_pack[r, sl] = ga[r, sl]
                tq_pack[r, pl.ds(64 + k * DH, DH)] = gb[r, sl]
                ts_pack[r, sl] = (gd[r, sl] - gc[r, sl]) * 0.25
                ts_pack[r, pl.ds(64 + k * DH, DH)] = ge[r, sl] - gf[r, sl]
                ts_pack[r, pl.ds(128 + k * DH, DH)] = gf[r, sl]
            return 0

        lax.fori_loop(0, NCH, row, 0)
        pltpu.sync_copy(tq_pack, tq_out.at[pl.ds(b, NCH)])
        pltpu.sync_copy(ts_pack, ts_out.at[pl.ds(b, NCH)])
        return 0

    lax.fori_loop(0, (NPAD // NWORK) // NCH, chunk, 0)


def _build_node_tables(qtab, ktab, vtab, iu0, iu1, ii0, ii1):
    mesh = plsc.VectorSubcoreMesh(core_axis_name="c", subcore_axis_name="s")
    f = pl.kernel(
        _stage1_body,
        out_type=[
            jax.ShapeDtypeStruct((NPAD, 2 * D), jnp.float32),
            jax.ShapeDtypeStruct((NPAD, 3 * D), jnp.float32),
        ],
        mesh=mesh,
        scratch_types=[
            pltpu.VMEM((NCH,), jnp.int32),
            pltpu.VMEM((NCH,), jnp.int32),
            pltpu.VMEM((NCH,), jnp.int32),
            pltpu.VMEM((NCH,), jnp.int32),
            pltpu.VMEM((NCH, D), jnp.float32),
            pltpu.VMEM((NCH, D), jnp.float32),
            pltpu.VMEM((NCH, D), jnp.float32),
            pltpu.VMEM((NCH, D), jnp.float32),
            pltpu.VMEM((NCH, D), jnp.float32),
            pltpu.VMEM((NCH, D), jnp.float32),
            pltpu.VMEM((NCH, 2 * D), jnp.float32),
            pltpu.VMEM((NCH, 3 * D), jnp.float32),
            pltpu.SemaphoreType.DMA,
        ],
        compiler_params=pltpu.CompilerParams(use_tc_tiling_on_sc=False, needs_layout_passes=False),
    )
    return f(qtab, ktab, vtab, iu0, iu1, ii0, ii1)


def _stage2_body(tq, ts, esrc, edst, out_hbm,
                 idx_s3, idx_d3, qbuf, sbuf, mbuf, acc, semg):
    cid = lax.axis_index("c")
    sid = lax.axis_index("s")
    rows_per_sub = NPAD // 16
    gbase = cid * (NCHUNK * 16) + sid * NCHUNK

    zero16 = jnp.zeros((DH,), jnp.float32)

    def zrow(r, _):
        for k in range(ROW // DH):
            mbuf[r, pl.ds(k * DH, DH)] = zero16
        return 0

    lax.fori_loop(0, ECH, zrow, 0)

    def zcopy(z, _):
        pltpu.sync_copy(mbuf.at[pl.ds(0, 32)],
                        acc.at[pl.ds(sid * rows_per_sub + z * 32, 32)])
        return 0

    lax.fori_loop(0, rows_per_sub // 32, zcopy, 0)
    plsc.subcore_barrier()

    one0 = jnp.where(lax.iota(jnp.int32, DH) == 0, 1.0, 0.0)

    def crow(r, _):
        mbuf[r, pl.ds(128, DH)] = one0
        return 0

    lax.fori_loop(0, ECH, crow, 0)

    def load_block(t, bp):
        pltpu.sync_copy(esrc.at[pl.ds(gbase + t * CPB, CPB)], idx_s3.at[bp])
        pltpu.sync_copy(edst.at[pl.ds(gbase + t * CPB, CPB)], idx_d3.at[bp])

    def issue_gathers(bp, row, slot):
        pltpu.async_copy(tq.at[idx_d3.at[bp, row]],
                         qbuf.at[pl.ds(slot, ECH)], semg)
        pltpu.async_copy(ts.at[idx_s3.at[bp, row]],
                         sbuf.at[pl.ds(slot, ECH)], semg)

    load_block(0, 0)
    issue_gathers(0, 0, 0)

    def chunk(j, _):
        slot = (j % 2) * ECH
        bp = (j // CPB) % 2
        row = j % CPB

        @pl.when(j < NCHUNK - 1)
        def _prefetch():
            nj = j + 1
            nbp = (nj // CPB) % 2
            nrow = nj % CPB

            @pl.when(nrow == 0)
            def _new_block():
                load_block(nj // CPB, nbp)

            issue_gathers(nbp, nrow, (nj % 2) * ECH)

        pltpu.make_async_copy(tq.at[idx_d3.at[bp, row]],
                              qbuf.at[pl.ds(slot, ECH)], semg).wait()
        pltpu.make_async_copy(ts.at[idx_s3.at[bp, row]],
                              sbuf.at[pl.ds(slot, ECH)], semg).wait()

        iota16 = lax.iota(jnp.int32, DH)
        zcol = jnp.zeros((DH,), jnp.int32)

        def group(g, _):
            erow = iota16 + (slot + g * DH)
            mrow = iota16 + g * DH
            for h in range(H):
                a0 = jnp.zeros((DH,), jnp.float32)
                a1 = jnp.zeros((DH,), jnp.float32)
                for jj in range(DH):
                    c0 = zcol + (h * DH + jj)
                    c1 = c0 + 64
                    dkj = plsc.load_gather(sbuf, [erow, c0])
                    q0j = plsc.load_gather(qbuf, [erow, c0])
                    q1j = plsc.load_gather(qbuf, [erow, c1])
                    a0 = a0 + q0j * dkj
                    a1 = a1 + q1j * dkj
                av0 = 1.0 / (1.0 + jnp.exp(a0))
                av1 = 1.0 / (1.0 + jnp.exp(a1))
                for jj in range(DH):
                    c0 = zcol + (h * DH + jj)
                    c1 = c0 + 64
                    c2 = c0 + 128
                    dvj = plsc.load_gather(sbuf, [erow, c1])
                    v1j = plsc.load_gather(sbuf, [erow, c2])
                    plsc.store_scatter(mbuf, [mrow, c0], v1j + av0 * dvj)
                    plsc.store_scatter(mbuf, [mrow, c1], v1j + av1 * dvj)
            return 0

        lax.fori_loop(0, ECH // DH, group, 0)
        pltpu.sync_copy(mbuf, acc.at[idx_d3.at[bp, row]], add=True)
        return 0

    lax.fori_loop(0, NCHUNK, chunk, 0)
    plsc.subcore_barrier()

    def ocopy(z, _):
        r0 = sid * rows_per_sub + z * 128
        pltpu.sync_copy(acc.at[pl.ds(r0, 128)], out_hbm.at[cid, pl.ds(r0, 128)])
        return 0

    lax.fori_loop(0, rows_per_sub // 128, ocopy, 0)


def _edge_aggregate(tq_rows, ts_rows, esrc, edst):
    mesh = plsc.VectorSubcoreMesh(core_axis_name="c", subcore_axis_name="s")
    f = pl.kernel(
        _stage2_body,
        out_type=jax.ShapeDtypeStruct((2, NPAD, ROW), jnp.float32),
        mesh=mesh,
        scratch_types=[
            pltpu.VMEM((2, CPB, ECH), jnp.int32),
            pltpu.VMEM((2, CPB, ECH), jnp.int32),
            pltpu.VMEM((2 * ECH, 2 * D), jnp.float32),
            pltpu.VMEM((2 * ECH, 3 * D), jnp.float32),
            pltpu.VMEM((ECH, ROW), jnp.float32),
            pltpu.VMEM_SHARED((NPAD, ROW), jnp.float32),
            pltpu.SemaphoreType.DMA,
        ],
        compiler_params=pltpu.CompilerParams(use_tc_tiling_on_sc=False, needs_layout_passes=False),
    )
    return f(tq_rows, ts_rows, esrc, edst)


def _epilogue_body(aggp, wo, gb_sc, wout, bout, out):
    a = aggp[0] + aggp[1]
    cnt = jnp.maximum(a[:N, 128:129], 1.0)
    y0 = jnp.dot(a[:N, 0:64], wo[...], preferred_element_type=jnp.float32) / cnt
    y1 = jnp.dot(a[:N, 64:128], wo[...], preferred_element_type=jnp.float32) / cnt
    inv = 1.0 / (N * D)
    m0 = jnp.sum(y0) * inv
    m1 = jnp.sum(y1) * inv
    v0 = jnp.sum(y0 * y0) * inv - m0 * m0
    v1 = jnp.sum(y1 * y1) * inv - m1 * m1
    g0, g1, b0, b1 = gb_sc[0], gb_sc[1], gb_sc[2], gb_sc[3]
    s0 = g0 * lax.rsqrt(v0 + 1e-5)
    s1 = g1 * lax.rsqrt(v1 + 1e-5)
    t = (b0 - s0 * m0) + (b1 - s1 * m1)
    z = s0 * y0 + s1 * y1 + t
    logits = jnp.dot(z, wout[...], preferred_element_type=jnp.float32) \
        + bout[...][None, :]
    mx = jnp.max(logits, axis=-1, keepdims=True)
    ex = jnp.exp(logits - mx)
    out[...] = ex / jnp.sum(ex, axis=-1, keepdims=True)


def _epilogue(aggp, Wo, gamma, beta, W_out, b_out):
    gb_sc = jnp.concatenate([gamma, beta]).astype(jnp.float32)
    return pl.pallas_call(
        _epilogue_body,
        out_shape=jax.ShapeDtypeStruct((N, NCLS), jnp.float32),
        in_specs=[
            pl.BlockSpec(memory_space=pltpu.VMEM),
            pl.BlockSpec(memory_space=pltpu.VMEM),
            pl.BlockSpec(memory_space=pltpu.SMEM),
            pl.BlockSpec(memory_space=pltpu.VMEM),
            pl.BlockSpec(memory_space=pltpu.VMEM),
        ],
    )(aggp, Wo, gb_sc, W_out, b_out)


def kernel(tf_users_cat, tf_items_cat, edge_index, emb_users, emb_items,
           Wq, Wk, Wv, Wo, gamma, beta, W_out, b_out):
    pe = _pos_encoding()
    qtab, ktab, vtab = _make_tables(emb_users, emb_items, Wq, Wk, Wv, pe)

    cu = tf_users_cat.astype(jnp.int32)
    ci = tf_items_cat.astype(jnp.int32)
    pad_n = NPAD - N
    iu0 = jnp.pad(cu[:, 0], (0, pad_n))
    iu1 = jnp.pad(cu[:, 1] + VOCAB, (0, pad_n))
    ii0 = jnp.pad(ci[:, 0], (0, pad_n))
    ii1 = jnp.pad(ci[:, 1] + VOCAB, (0, pad_n))

    tq_rows, ts_rows = _build_node_tables(qtab, ktab, vtab, iu0, iu1, ii0, ii1)

    ei = edge_index.astype(jnp.int32)
    esrc = jnp.pad(ei[0], (0, EPAD - E)).reshape(NWORK * NCHUNK, ECH)
    edst = jnp.pad(ei[1], (0, EPAD - E),
                   constant_values=N).reshape(NWORK * NCHUNK, ECH)

    aggp = _edge_aggregate(tq_rows, ts_rows, esrc, edst)
    return _epilogue(aggp, Wo, gamma, beta, W_out, b_out)


# speedup vs baseline: 2.5280x; 2.5280x over previous
import functools
import numpy as np
import jax
import jax.numpy as jnp
from jax import lax
from jax.experimental import pallas as pl
from jax.experimental.pallas import tpu as pltpu
from jax.experimental.pallas import tpu_sc as plsc

D = 64
C = 2
H = 4
DH = 16
NCLS = 8
N = 10000
E = 160000
VOCAB = 1000

NWORK = 32
NPAD = 10240
ECH = 32
CPB = 8
NCHUNK = 160
NCH = 64
EPAD = NWORK * NCHUNK * ECH
ROW = 144


def _pos_encoding():
    pos = np.arange(C)[:, None].astype(np.float64)
    i = np.arange(D)[None, :].astype(np.float64)
    angle = pos / np.power(10000.0, (2.0 * (i // 2)) / D)
    pe = np.where(i % 2 == 0, np.sin(angle), np.cos(angle))
    return jnp.asarray(pe, dtype=jnp.float32)


def _tables_body(eu, ei, wq, wk, wv, pe, qt, kt, vt):
    for c in range(C):
        xu = eu[c] + pe[c][None, :]
        xi = ei[c] + pe[c][None, :]
        qt[c * VOCAB:(c + 1) * VOCAB, :] = jnp.dot(
            xu, wq[...], preferred_element_type=jnp.float32)
        kt[c * VOCAB:(c + 1) * VOCAB, :] = jnp.dot(
            xi, wk[...], preferred_element_type=jnp.float32)
        vt[c * VOCAB:(c + 1) * VOCAB, :] = jnp.dot(
            xi, wv[...], preferred_element_type=jnp.float32)


def _make_tables(emb_users, emb_items, Wq, Wk, Wv, pe):
    out = [jax.ShapeDtypeStruct((C * VOCAB, D), jnp.float32)] * 3
    return pl.pallas_call(_tables_body, out_shape=out)(
        emb_users, emb_items, Wq, Wk, Wv, pe)


def _stage1_body(qtab, ktab, vtab, iu0, iu1, ii0, ii1, tq_out, ts_out,
                 vi0, vi1, vi2, vi3, ga, gb, gc, gd, ge, gf,
                 tq_pack, ts_pack, sem):
    cid = lax.axis_index("c")
    sid = lax.axis_index("s")
    wid = sid * 2 + cid
    base = wid * (NPAD // NWORK)

    def chunk(j, _):
        b = base + j * NCH
        pltpu.sync_copy(iu0.at[pl.ds(b, NCH)], vi0)
        pltpu.sync_copy(iu1.at[pl.ds(b, NCH)], vi1)
        pltpu.sync_copy(ii0.at[pl.ds(b, NCH)], vi2)
        pltpu.sync_copy(ii1.at[pl.ds(b, NCH)], vi3)
        pltpu.async_copy(qtab.at[vi0], ga, sem).wait()
        pltpu.async_copy(qtab.at[vi1], gb, sem).wait()
        pltpu.async_copy(ktab.at[vi2], gc, sem).wait()
        pltpu.async_copy(ktab.at[vi3], gd, sem).wait()
        pltpu.async_copy(vtab.at[vi2], ge, sem).wait()
        pltpu.async_copy(vtab.at[vi3], gf, sem).wait()

        def row(r, _):
            for k in range(4):
                sl = pl.ds(k * DH, DH)
                tq_pack[r, sl] = ga[r, sl]
                tq_pack[r, pl.ds(64 + k * DH, DH)] = gb[r, sl]
                ts_pack[r, sl] = (gd[r, sl] - gc[r, sl]) * 0.25
                ts_pack[r, pl.ds(64 + k * DH, DH)] = ge[r, sl] - gf[r, sl]
                ts_pack[r, pl.ds(128 + k * DH, DH)] = gf[r, sl]
            return 0

        lax.fori_loop(0, NCH, row, 0)
        pltpu.sync_copy(tq_pack, tq_out.at[pl.ds(b, NCH)])
        pltpu.sync_copy(ts_pack, ts_out.at[pl.ds(b, NCH)])
        return 0

    lax.fori_loop(0, (NPAD // NWORK) // NCH, chunk, 0)


def _build_node_tables(qtab, ktab, vtab, iu0, iu1, ii0, ii1):
    mesh = plsc.VectorSubcoreMesh(core_axis_name="c", subcore_axis_name="s")
    f = pl.kernel(
        _stage1_body,
        out_type=[
            jax.ShapeDtypeStruct((NPAD, 2 * D), jnp.float32),
            jax.ShapeDtypeStruct((NPAD, 3 * D), jnp.float32),
        ],
        mesh=mesh,
        scratch_types=[
            pltpu.VMEM((NCH,), jnp.int32),
            pltpu.VMEM((NCH,), jnp.int32),
            pltpu.VMEM((NCH,), jnp.int32),
            pltpu.VMEM((NCH,), jnp.int32),
            pltpu.VMEM((NCH, D), jnp.float32),
            pltpu.VMEM((NCH, D), jnp.float32),
            pltpu.VMEM((NCH, D), jnp.float32),
            pltpu.VMEM((NCH, D), jnp.float32),
            pltpu.VMEM((NCH, D), jnp.float32),
            pltpu.VMEM((NCH, D), jnp.float32),
            pltpu.VMEM((NCH, 2 * D), jnp.float32),
            pltpu.VMEM((NCH, 3 * D), jnp.float32),
            pltpu.SemaphoreType.DMA,
        ],
        compiler_params=pltpu.CompilerParams(use_tc_tiling_on_sc=False, needs_layout_passes=False),
    )
    return f(qtab, ktab, vtab, iu0, iu1, ii0, ii1)


def _stage2_body(tq, ts, esrc, edst, out_hbm,
                 idx_s3, idx_d3, qbuf, sbuf, mbuf, acc, semg):
    cid = lax.axis_index("c")
    sid = lax.axis_index("s")
    rows_per_sub = NPAD // 16
    gbase = cid * (NCHUNK * 16) + sid * NCHUNK

    zero16 = jnp.zeros((DH,), jnp.float32)

    def zrow(r, _):
        for k in range(ROW // DH):
            mbuf[r, pl.ds(k * DH, DH)] = zero16
        return 0

    lax.fori_loop(0, ECH, zrow, 0)

    def zcopy(z, _):
        pltpu.sync_copy(mbuf.at[pl.ds(0, 32)],
                        acc.at[pl.ds(sid * rows_per_sub + z * 32, 32)])
        return 0

    lax.fori_loop(0, rows_per_sub // 32, zcopy, 0)
    plsc.subcore_barrier()

    one0 = jnp.where(lax.iota(jnp.int32, DH) == 0, 1.0, 0.0)

    def crow(r, _):
        mbuf[r, pl.ds(128, DH)] = one0
        return 0

    lax.fori_loop(0, ECH, crow, 0)

    def load_block(t, bp):
        pltpu.sync_copy(esrc.at[pl.ds(gbase + t * CPB, CPB)], idx_s3.at[bp])
        pltpu.sync_copy(edst.at[pl.ds(gbase + t * CPB, CPB)], idx_d3.at[bp])

    def issue_gathers(bp, row, slot):
        pltpu.async_copy(tq.at[idx_d3.at[bp, row]],
                         qbuf.at[pl.ds(slot, ECH)], semg)
        pltpu.async_copy(ts.at[idx_s3.at[bp, row]],
                         sbuf.at[pl.ds(slot, ECH)], semg)

    load_block(0, 0)
    issue_gathers(0, 0, 0)

    def chunk(j, _):
        slot = (j % 2) * ECH
        bp = (j // CPB) % 2
        row = j % CPB

        @pl.when(j < NCHUNK - 1)
        def _prefetch():
            nj = j + 1
            nbp = (nj // CPB) % 2
            nrow = nj % CPB

            @pl.when(nrow == 0)
            def _new_block():
                load_block(nj // CPB, nbp)

            issue_gathers(nbp, nrow, (nj % 2) * ECH)

        pltpu.make_async_copy(tq.at[idx_d3.at[bp, row]],
                              qbuf.at[pl.ds(slot, ECH)], semg).wait()
        pltpu.make_async_copy(ts.at[idx_s3.at[bp, row]],
                              sbuf.at[pl.ds(slot, ECH)], semg).wait()

        iota16 = lax.iota(jnp.int32, DH)
        zcol = jnp.zeros((DH,), jnp.int32)

        def group(g, _):
            erow = iota16 + (slot + g * DH)
            mrow = iota16 + g * DH
            for h in range(H):
                def dotstep(t, carry):
                    a0, a1 = carry
                    for u in range(4):
                        c0 = ((iota16 + (t * 4 + u)) & 15) + (h * DH)
                        c1 = c0 + 64
                        dkj = plsc.load_gather(sbuf, [erow, c0])
                        q0j = plsc.load_gather(qbuf, [erow, c0])
                        q1j = plsc.load_gather(qbuf, [erow, c1])
                        a0 = a0 + q0j * dkj
                        a1 = a1 + q1j * dkj
                    return a0, a1

                a0, a1 = lax.fori_loop(0, DH // 4, dotstep,
                                       (jnp.zeros((DH,), jnp.float32),
                                        jnp.zeros((DH,), jnp.float32)))
                av0 = 1.0 / (1.0 + jnp.exp(a0))
                av1 = 1.0 / (1.0 + jnp.exp(a1))

                def combstep(t, _):
                    for u in range(4):
                        c0 = ((iota16 + (t * 4 + u)) & 15) + (h * DH)
                        c1 = c0 + 64
                        c2 = c0 + 128
                        dvj = plsc.load_gather(sbuf, [erow, c1])
                        v1j = plsc.load_gather(sbuf, [erow, c2])
                        plsc.store_scatter(mbuf, [mrow, c0], v1j + av0 * dvj)
                        plsc.store_scatter(mbuf, [mrow, c1], v1j + av1 * dvj)
                    return 0

                lax.fori_loop(0, DH // 4, combstep, 0)
            return 0

        lax.fori_loop(0, ECH // DH, group, 0)
        pltpu.sync_copy(mbuf, acc.at[idx_d3.at[bp, row]], add=True)
        return 0

    lax.fori_loop(0, NCHUNK, chunk, 0)
    plsc.subcore_barrier()

    def ocopy(z, _):
        r0 = sid * rows_per_sub + z * 128
        pltpu.sync_copy(acc.at[pl.ds(r0, 128)], out_hbm.at[cid, pl.ds(r0, 128)])
        return 0

    lax.fori_loop(0, rows_per_sub // 128, ocopy, 0)


def _edge_aggregate(tq_rows, ts_rows, esrc, edst):
    mesh = plsc.VectorSubcoreMesh(core_axis_name="c", subcore_axis_name="s")
    f = pl.kernel(
        _stage2_body,
        out_type=jax.ShapeDtypeStruct((2, NPAD, ROW), jnp.float32),
        mesh=mesh,
        scratch_types=[
            pltpu.VMEM((2, CPB, ECH), jnp.int32),
            pltpu.VMEM((2, CPB, ECH), jnp.int32),
            pltpu.VMEM((2 * ECH, 2 * D), jnp.float32),
            pltpu.VMEM((2 * ECH, 3 * D), jnp.float32),
            pltpu.VMEM((ECH, ROW), jnp.float32),
            pltpu.VMEM_SHARED((NPAD, ROW), jnp.float32),
            pltpu.SemaphoreType.DMA,
        ],
        compiler_params=pltpu.CompilerParams(use_tc_tiling_on_sc=False, needs_layout_passes=False),
    )
    return f(tq_rows, ts_rows, esrc, edst)


def _epilogue_body(aggp, wo, gb_sc, wout, bout, out):
    a = aggp[0] + aggp[1]
    cnt = jnp.maximum(a[:N, 128:129], 1.0)
    y0 = jnp.dot(a[:N, 0:64], wo[...], preferred_element_type=jnp.float32) / cnt
    y1 = jnp.dot(a[:N, 64:128], wo[...], preferred_element_type=jnp.float32) / cnt
    inv = 1.0 / (N * D)
    m0 = jnp.sum(y0) * inv
    m1 = jnp.sum(y1) * inv
    v0 = jnp.sum(y0 * y0) * inv - m0 * m0
    v1 = jnp.sum(y1 * y1) * inv - m1 * m1
    g0, g1, b0, b1 = gb_sc[0], gb_sc[1], gb_sc[2], gb_sc[3]
    s0 = g0 * lax.rsqrt(v0 + 1e-5)
    s1 = g1 * lax.rsqrt(v1 + 1e-5)
    t = (b0 - s0 * m0) + (b1 - s1 * m1)
    z = s0 * y0 + s1 * y1 + t
    logits = jnp.dot(z, wout[...], preferred_element_type=jnp.float32) \
        + bout[...][None, :]
    mx = jnp.max(logits, axis=-1, keepdims=True)
    ex = jnp.exp(logits - mx)
    out[...] = ex / jnp.sum(ex, axis=-1, keepdims=True)


def _epilogue(aggp, Wo, gamma, beta, W_out, b_out):
    gb_sc = jnp.concatenate([gamma, beta]).astype(jnp.float32)
    return pl.pallas_call(
        _epilogue_body,
        out_shape=jax.ShapeDtypeStruct((N, NCLS), jnp.float32),
        in_specs=[
            pl.BlockSpec(memory_space=pltpu.VMEM),
            pl.BlockSpec(memory_space=pltpu.VMEM),
            pl.BlockSpec(memory_space=pltpu.SMEM),
            pl.BlockSpec(memory_space=pltpu.VMEM),
            pl.BlockSpec(memory_space=pltpu.VMEM),
        ],
    )(aggp, Wo, gb_sc, W_out, b_out)


def kernel(tf_users_cat, tf_items_cat, edge_index, emb_users, emb_items,
           Wq, Wk, Wv, Wo, gamma, beta, W_out, b_out):
    pe = _pos_encoding()
    qtab, ktab, vtab = _make_tables(emb_users, emb_items, Wq, Wk, Wv, pe)

    cu = tf_users_cat.astype(jnp.int32)
    ci = tf_items_cat.astype(jnp.int32)
    pad_n = NPAD - N
    iu0 = jnp.pad(cu[:, 0], (0, pad_n))
    iu1 = jnp.pad(cu[:, 1] + VOCAB, (0, pad_n))
    ii0 = jnp.pad(ci[:, 0], (0, pad_n))
    ii1 = jnp.pad(ci[:, 1] + VOCAB, (0, pad_n))

    tq_rows, ts_rows = _build_node_tables(qtab, ktab, vtab, iu0, iu1, ii0, ii1)

    ei = edge_index.astype(jnp.int32)
    esrc = jnp.pad(ei[0], (0, EPAD - E)).reshape(NWORK * NCHUNK, ECH)
    edst = jnp.pad(ei[1], (0, EPAD - E),
                   constant_values=N).reshape(NWORK * NCHUNK, ECH)

    aggp = _edge_aggregate(tq_rows, ts_rows, esrc, edst)
    return _epilogue(aggp, Wo, gamma, beta, W_out, b_out)


# speedup vs baseline: 2.8080x; 1.1108x over previous
import functools
import numpy as np
import jax
import jax.numpy as jnp
from jax import lax
from jax.experimental import pallas as pl
from jax.experimental.pallas import tpu as pltpu
from jax.experimental.pallas import tpu_sc as plsc

D = 64
C = 2
H = 4
DH = 16
NCLS = 8
N = 10000
E = 160000
VOCAB = 1000

NWORK = 32
NPAD = 10240
ECH = 32
CPB = 8
NCHUNK = 160
NCH = 64
EPAD = NWORK * NCHUNK * ECH
ROW = 144


def _pos_encoding():
    pos = np.arange(C)[:, None].astype(np.float64)
    i = np.arange(D)[None, :].astype(np.float64)
    angle = pos / np.power(10000.0, (2.0 * (i // 2)) / D)
    pe = np.where(i % 2 == 0, np.sin(angle), np.cos(angle))
    return jnp.asarray(pe, dtype=jnp.float32)


def _tables_body(eu, ei, wq, wk, wv, pe, qt, kt, vt):
    for c in range(C):
        xu = eu[c] + pe[c][None, :]
        xi = ei[c] + pe[c][None, :]
        qt[c * VOCAB:(c + 1) * VOCAB, :] = jnp.dot(
            xu, wq[...], preferred_element_type=jnp.float32)
        kt[c * VOCAB:(c + 1) * VOCAB, :] = jnp.dot(
            xi, wk[...], preferred_element_type=jnp.float32)
        vt[c * VOCAB:(c + 1) * VOCAB, :] = jnp.dot(
            xi, wv[...], preferred_element_type=jnp.float32)


def _make_tables(emb_users, emb_items, Wq, Wk, Wv, pe):
    out = [jax.ShapeDtypeStruct((C * VOCAB, D), jnp.float32)] * 3
    return pl.pallas_call(_tables_body, out_shape=out)(
        emb_users, emb_items, Wq, Wk, Wv, pe)


def _stage1_body(qtab, ktab, vtab, iu0, iu1, ii0, ii1, tq_out, ts_out,
                 vi0, vi1, vi2, vi3, ga, gb, gc, gd, ge, gf,
                 tq_pack, ts_pack, sem):
    cid = lax.axis_index("c")
    sid = lax.axis_index("s")
    wid = sid * 2 + cid
    base = wid * (NPAD // NWORK)

    def chunk(j, _):
        b = base + j * NCH
        pltpu.sync_copy(iu0.at[pl.ds(b, NCH)], vi0)
        pltpu.sync_copy(iu1.at[pl.ds(b, NCH)], vi1)
        pltpu.sync_copy(ii0.at[pl.ds(b, NCH)], vi2)
        pltpu.sync_copy(ii1.at[pl.ds(b, NCH)], vi3)
        cps = [pltpu.async_copy(qtab.at[vi0], ga, sem),
               pltpu.async_copy(qtab.at[vi1], gb, sem),
               pltpu.async_copy(ktab.at[vi2], gc, sem),
               pltpu.async_copy(ktab.at[vi3], gd, sem),
               pltpu.async_copy(vtab.at[vi2], ge, sem),
               pltpu.async_copy(vtab.at[vi3], gf, sem)]
        for cp in cps:
            cp.wait()

        def row(r, _):
            for k in range(4):
                sl = pl.ds(k * DH, DH)
                tq_pack[r, sl] = ga[r, sl]
                tq_pack[r, pl.ds(64 + k * DH, DH)] = gb[r, sl]
                ts_pack[r, sl] = (gd[r, sl] - gc[r, sl]) * 0.25
                ts_pack[r, pl.ds(64 + k * DH, DH)] = ge[r, sl] - gf[r, sl]
                ts_pack[r, pl.ds(128 + k * DH, DH)] = gf[r, sl]
            return 0

        lax.fori_loop(0, NCH, row, 0)
        pltpu.sync_copy(tq_pack, tq_out.at[pl.ds(b, NCH)])
        pltpu.sync_copy(ts_pack, ts_out.at[pl.ds(b, NCH)])
        return 0

    lax.fori_loop(0, (NPAD // NWORK) // NCH, chunk, 0)


def _build_node_tables(qtab, ktab, vtab, iu0, iu1, ii0, ii1):
    mesh = plsc.VectorSubcoreMesh(core_axis_name="c", subcore_axis_name="s")
    f = pl.kernel(
        _stage1_body,
        out_type=[
            jax.ShapeDtypeStruct((NPAD, 2 * D), jnp.float32),
            jax.ShapeDtypeStruct((NPAD, 3 * D), jnp.float32),
        ],
        mesh=mesh,
        scratch_types=[
            pltpu.VMEM((NCH,), jnp.int32),
            pltpu.VMEM((NCH,), jnp.int32),
            pltpu.VMEM((NCH,), jnp.int32),
            pltpu.VMEM((NCH,), jnp.int32),
            pltpu.VMEM((NCH, D), jnp.float32),
            pltpu.VMEM((NCH, D), jnp.float32),
            pltpu.VMEM((NCH, D), jnp.float32),
            pltpu.VMEM((NCH, D), jnp.float32),
            pltpu.VMEM((NCH, D), jnp.float32),
            pltpu.VMEM((NCH, D), jnp.float32),
            pltpu.VMEM((NCH, 2 * D), jnp.float32),
            pltpu.VMEM((NCH, 3 * D), jnp.float32),
            pltpu.SemaphoreType.DMA,
        ],
        compiler_params=pltpu.CompilerParams(use_tc_tiling_on_sc=False, needs_layout_passes=False),
    )
    return f(qtab, ktab, vtab, iu0, iu1, ii0, ii1)


def _stage2_body(tq, ts, esrc, edst, out_hbm,
                 idx_s3, idx_d3, qbuf, sbuf, mbuf, acc, semg, semsc):
    cid = lax.axis_index("c")
    sid = lax.axis_index("s")
    rows_per_sub = NPAD // 16
    gbase = cid * (NCHUNK * 16) + sid * NCHUNK

    zero16 = jnp.zeros((DH,), jnp.float32)

    def zrow(r, _):
        for k in range(ROW // DH):
            mbuf[0, r, pl.ds(k * DH, DH)] = zero16
        return 0

    lax.fori_loop(0, ECH, zrow, 0)

    def zcopy(z, _):
        pltpu.sync_copy(mbuf.at[0],
                        acc.at[pl.ds(sid * rows_per_sub + z * ECH, ECH)])
        return 0

    lax.fori_loop(0, rows_per_sub // ECH, zcopy, 0)
    plsc.subcore_barrier()

    one0 = jnp.where(lax.iota(jnp.int32, DH) == 0, 1.0, 0.0)

    def crow(r, _):
        mbuf[0, r, pl.ds(128, DH)] = one0
        mbuf[1, r, pl.ds(128, DH)] = one0
        return 0

    lax.fori_loop(0, ECH, crow, 0)

    def load_block(t, bp):
        pltpu.sync_copy(esrc.at[pl.ds(gbase + t * CPB, CPB)], idx_s3.at[bp])
        pltpu.sync_copy(edst.at[pl.ds(gbase + t * CPB, CPB)], idx_d3.at[bp])

    def issue_gathers(bp, row, slot):
        pltpu.async_copy(tq.at[idx_d3.at[bp, row]],
                         qbuf.at[pl.ds(slot, ECH)], semg)
        pltpu.async_copy(ts.at[idx_s3.at[bp, row]],
                         sbuf.at[pl.ds(slot, ECH)], semg)

    load_block(0, 0)
    issue_gathers(0, 0, 0)

    def chunk(j, _):
        slot = (j % 2) * ECH
        ms = j % 2
        bp = (j // CPB) % 2
        row = j % CPB

        @pl.when(j < NCHUNK - 1)
        def _prefetch():
            nj = j + 1
            nbp = (nj // CPB) % 2
            nrow = nj % CPB

            @pl.when(nrow == 0)
            def _new_block():
                load_block(nj // CPB, nbp)

            issue_gathers(nbp, nrow, (nj % 2) * ECH)

        pltpu.make_async_copy(tq.at[idx_d3.at[bp, row]],
                              qbuf.at[pl.ds(slot, ECH)], semg).wait()
        pltpu.make_async_copy(ts.at[idx_s3.at[bp, row]],
                              sbuf.at[pl.ds(slot, ECH)], semg).wait()

        @pl.when(j >= 2)
        def _wait_scatter():
            pltpu.make_async_copy(mbuf.at[ms], acc.at[idx_d3.at[bp, row]],
                                  semsc).wait()

        iota16 = lax.iota(jnp.int32, DH)
        zcol = jnp.zeros((DH,), jnp.int32)

        def group(g, _):
            erow = iota16 + (slot + g * DH)
            mrow = iota16 + g * DH
            for h in range(H):
                def dotstep(t, carry):
                    a0, a1 = carry
                    for u in range(4):
                        c0 = ((iota16 + (t * 4 + u)) & 15) + (h * DH)
                        c1 = c0 + 64
                        dkj = plsc.load_gather(sbuf, [erow, c0])
                        q0j = plsc.load_gather(qbuf, [erow, c0])
                        q1j = plsc.load_gather(qbuf, [erow, c1])
                        a0 = a0 + q0j * dkj
                        a1 = a1 + q1j * dkj
                    return a0, a1

                a0, a1 = lax.fori_loop(0, DH // 4, dotstep,
                                       (jnp.zeros((DH,), jnp.float32),
                                        jnp.zeros((DH,), jnp.float32)))
                av0 = 1.0 / (1.0 + jnp.exp(a0))
                av1 = 1.0 / (1.0 + jnp.exp(a1))

                def combstep(t, _):
                    for u in range(4):
                        c0 = ((iota16 + (t * 4 + u)) & 15) + (h * DH)
                        c1 = c0 + 64
                        c2 = c0 + 128
                        dvj = plsc.load_gather(sbuf, [erow, c1])
                        v1j = plsc.load_gather(sbuf, [erow, c2])
                        plsc.store_scatter(mbuf.at[ms], [mrow, c0],
                                           v1j + av0 * dvj)
                        plsc.store_scatter(mbuf.at[ms], [mrow, c1],
                                           v1j + av1 * dvj)
                    return 0

                lax.fori_loop(0, DH // 4, combstep, 0)
            return 0

        lax.fori_loop(0, ECH // DH, group, 0)
        pltpu.async_copy(mbuf.at[ms], acc.at[idx_d3.at[bp, row]], semsc,
                         add=True)
        return 0

    lax.fori_loop(0, NCHUNK, chunk, 0)
    for tail in (NCHUNK - 2, NCHUNK - 1):
        pltpu.make_async_copy(mbuf.at[tail % 2], acc.at[idx_d3.at[0, 0]],
                              semsc).wait()
    plsc.subcore_barrier()

    def ocopy(z, _):
        r0 = sid * rows_per_sub + z * 128
        pltpu.sync_copy(acc.at[pl.ds(r0, 128)], out_hbm.at[cid, pl.ds(r0, 128)])
        return 0

    lax.fori_loop(0, rows_per_sub // 128, ocopy, 0)


def _edge_aggregate(tq_rows, ts_rows, esrc, edst):
    mesh = plsc.VectorSubcoreMesh(core_axis_name="c", subcore_axis_name="s")
    f = pl.kernel(
        _stage2_body,
        out_type=jax.ShapeDtypeStruct((2, NPAD, ROW), jnp.float32),
        mesh=mesh,
        scratch_types=[
            pltpu.VMEM((2, CPB, ECH), jnp.int32),
            pltpu.VMEM((2, CPB, ECH), jnp.int32),
            pltpu.VMEM((2 * ECH, 2 * D), jnp.float32),
            pltpu.VMEM((2 * ECH, 3 * D), jnp.float32),
            pltpu.VMEM((2, ECH, ROW), jnp.float32),
            pltpu.VMEM_SHARED((NPAD, ROW), jnp.float32),
            pltpu.SemaphoreType.DMA,
            pltpu.SemaphoreType.DMA,
        ],
        compiler_params=pltpu.CompilerParams(use_tc_tiling_on_sc=False, needs_layout_passes=False),
    )
    return f(tq_rows, ts_rows, esrc, edst)


def _epilogue_body(aggp, wo, gb_sc, wout, bout, out):
    a = aggp[0] + aggp[1]
    cnt = jnp.maximum(a[:N, 128:129], 1.0)
    y0 = jnp.dot(a[:N, 0:64], wo[...], preferred_element_type=jnp.float32) / cnt
    y1 = jnp.dot(a[:N, 64:128], wo[...], preferred_element_type=jnp.float32) / cnt
    inv = 1.0 / (N * D)
    m0 = jnp.sum(y0) * inv
    m1 = jnp.sum(y1) * inv
    v0 = jnp.sum(y0 * y0) * inv - m0 * m0
    v1 = jnp.sum(y1 * y1) * inv - m1 * m1
    g0, g1, b0, b1 = gb_sc[0], gb_sc[1], gb_sc[2], gb_sc[3]
    s0 = g0 * lax.rsqrt(v0 + 1e-5)
    s1 = g1 * lax.rsqrt(v1 + 1e-5)
    t = (b0 - s0 * m0) + (b1 - s1 * m1)
    z = s0 * y0 + s1 * y1 + t
    logits = jnp.dot(z, wout[...], preferred_element_type=jnp.float32) \
        + bout[...][None, :]
    mx = jnp.max(logits, axis=-1, keepdims=True)
    ex = jnp.exp(logits - mx)
    out[...] = ex / jnp.sum(ex, axis=-1, keepdims=True)


def _epilogue(aggp, Wo, gamma, beta, W_out, b_out):
    gb_sc = jnp.concatenate([gamma, beta]).astype(jnp.float32)
    return pl.pallas_call(
        _epilogue_body,
        out_shape=jax.ShapeDtypeStruct((N, NCLS), jnp.float32),
        in_specs=[
            pl.BlockSpec(memory_space=pltpu.VMEM),
            pl.BlockSpec(memory_space=pltpu.VMEM),
            pl.BlockSpec(memory_space=pltpu.SMEM),
            pl.BlockSpec(memory_space=pltpu.VMEM),
            pl.BlockSpec(memory_space=pltpu.VMEM),
        ],
    )(aggp, Wo, gb_sc, W_out, b_out)


def kernel(tf_users_cat, tf_items_cat, edge_index, emb_users, emb_items,
           Wq, Wk, Wv, Wo, gamma, beta, W_out, b_out):
    pe = _pos_encoding()
    qtab, ktab, vtab = _make_tables(emb_users, emb_items, Wq, Wk, Wv, pe)

    cu = tf_users_cat.astype(jnp.int32)
    ci = tf_items_cat.astype(jnp.int32)
    pad_n = NPAD - N
    iu0 = jnp.pad(cu[:, 0], (0, pad_n))
    iu1 = jnp.pad(cu[:, 1] + VOCAB, (0, pad_n))
    ii0 = jnp.pad(ci[:, 0], (0, pad_n))
    ii1 = jnp.pad(ci[:, 1] + VOCAB, (0, pad_n))

    tq_rows, ts_rows = _build_node_tables(qtab, ktab, vtab, iu0, iu1, ii0, ii1)

    ei = edge_index.astype(jnp.int32)
    esrc = jnp.pad(ei[0], (0, EPAD - E)).reshape(NWORK * NCHUNK, ECH)
    edst = jnp.pad(ei[1], (0, EPAD - E),
                   constant_values=N).reshape(NWORK * NCHUNK, ECH)

    aggp = _edge_aggregate(tq_rows, ts_rows, esrc, edst)
    return _epilogue(aggp, Wo, gamma, beta, W_out, b_out)
